# baseline (speedup 1.0000x reference)
import jax
import jax.numpy as jnp
from jax.experimental import pallas as pl
from jax.experimental.pallas import tpu as pltpu

_NUM_SAMPLES = 100000
_NUM_FEATURES = 32
_BATCH = 1024
_INV_TEMP = 20.0

_BAND = 32
_NBANDS = _BATCH // _BAND
_CHUNK = 25088
_NBUF = 2
_NCORES = 4


def kernel(inputs, targets, features):
    del targets
    features_t = jnp.swapaxes(features, 0, 1)
    mesh = pltpu.create_tensorcore_mesh("core", num_cores=_NCORES)
    ncores = _NCORES

    def owner(b):
        return b % ncores

    def body(x_hbm, ft_hbm, o_hbm, xvm, ftvm, obuf, *sems):
        core = jax.lax.axis_index("core")
        store_sem = sems[:_NBUF]
        xf_sem = sems[_NBUF:]

        x_load = pltpu.make_async_copy(x_hbm, xvm, xf_sem[0])
        ft_load = pltpu.make_async_copy(ft_hbm, ftvm, xf_sem[1])
        x_load.start()
        ft_load.start()
        x_load.wait()
        ft_load.wait()

        x = xvm[...]
        nrm = jnp.sqrt(jnp.sum(x * x, axis=1, keepdims=True))
        xn = x * (_INV_TEMP / jnp.clip(nrm, 1e-12, None))

        def store(b):
            slot = (b // ncores) % _NBUF
            return pltpu.make_async_copy(
                obuf.at[slot],
                o_hbm.at[pl.ds(b * _BAND, _BAND)],
                store_sem[slot])

        for b in range(_NBANDS):
            @pl.when(owner(b) == core)
            def _(b=b):
                slot = (b // ncores) % _NBUF
                if b - ncores * _NBUF >= 0:
                    store(b - ncores * _NBUF).wait()
                xb = xn[b * _BAND:(b + 1) * _BAND]
                c = 0
                while c < _NUM_SAMPLES:
                    w = min(_CHUNK, _NUM_SAMPLES - c)
                    obuf[slot, :, c:c + w] = jax.lax.dot_general(
                        xb, ftvm[:, c:c + w], (((1,), (0,)), ((), ())),
                        preferred_element_type=jnp.float32)
                    c += _CHUNK
                store(b).start()

        for b in range(max(0, _NBANDS - ncores * _NBUF), _NBANDS):
            @pl.when(owner(b) == core)
            def _(b=b):
                store(b).wait()

    run = pl.kernel(
        body,
        out_type=jax.ShapeDtypeStruct((_BATCH, _NUM_SAMPLES), jnp.float32),
        mesh=mesh,
        scratch_types=[
            pltpu.VMEM((_BATCH, _NUM_FEATURES), jnp.float32),
            pltpu.VMEM((_NUM_FEATURES, _NUM_SAMPLES), jnp.float32),
            pltpu.VMEM((_NBUF, _BAND, _NUM_SAMPLES), jnp.float32),
        ] + [pltpu.SemaphoreType.DMA] * (_NBUF + 2),
    )
    return run(inputs, features_t)


# speedup vs baseline: 1.0950x; 1.0950x over previous
import jax
import jax.numpy as jnp
from jax.experimental import pallas as pl
from jax.experimental.pallas import tpu as pltpu

_NUM_SAMPLES = 100000
_NUM_FEATURES = 32
_BATCH = 1024
_INV_TEMP = 20.0

_BAND = 16
_NBANDS = _BATCH // _BAND
_CHUNK = 25088
_NSLOT = 6
_SUBROWS = 4
_NSUB = _BAND // _SUBROWS


def _body(x_ref, ft_ref, o_hbm, obuf, *sems):
    x = x_ref[...]
    nrm = jnp.sqrt(jnp.sum(x * x, axis=1, keepdims=True))
    xn = x * (_INV_TEMP / jnp.clip(nrm, 1e-12, None))

    def store(b, p):
        slot = b % _NSLOT
        return pltpu.make_async_copy(
            obuf.at[slot, pl.ds(p * _SUBROWS, _SUBROWS)],
            o_hbm.at[pl.ds(b * _BAND + p * _SUBROWS, _SUBROWS)],
            sems[slot * _NSUB + p])

    for b in range(_NBANDS):
        slot = b % _NSLOT
        if b >= _NSLOT:
            for p in range(_NSUB):
                store(b - _NSLOT, p).wait()
        xb = xn[b * _BAND:(b + 1) * _BAND]
        c = 0
        while c < _NUM_SAMPLES:
            w = min(_CHUNK, _NUM_SAMPLES - c)
            obuf[slot, :, c:c + w] = jax.lax.dot_general(
                xb, ft_ref[:, c:c + w], (((1,), (0,)), ((), ())),
                preferred_element_type=jnp.float32)
            c += _CHUNK
        for p in range(_NSUB):
            store(b, p).start()

    for b in range(_NBANDS - _NSLOT, _NBANDS):
        for p in range(_NSUB):
            store(b, p).wait()


def kernel(inputs, targets, features):
    del targets
    features_t = jnp.swapaxes(features, 0, 1)
    return pl.pallas_call(
        _body,
        in_specs=[
            pl.BlockSpec((_BATCH, _NUM_FEATURES), lambda: (0, 0)),
            pl.BlockSpec(memory_space=pltpu.MemorySpace.VMEM),
        ],
        out_specs=pl.BlockSpec(memory_space=pltpu.MemorySpace.HBM),
        out_shape=jax.ShapeDtypeStruct((_BATCH, _NUM_SAMPLES), jnp.float32),
        scratch_shapes=[
            pltpu.VMEM((_NSLOT, _BAND, _NUM_SAMPLES), jnp.float32),
        ] + [pltpu.SemaphoreType.DMA] * (_NSLOT * _NSUB),
    )(inputs, features_t)


# speedup vs baseline: 1.0981x; 1.0028x over previous
import jax
import jax.numpy as jnp
from jax.experimental import pallas as pl
from jax.experimental.pallas import tpu as pltpu

_NUM_SAMPLES = 100000
_NUM_FEATURES = 32
_BATCH = 1024
_INV_TEMP = 20.0

_BAND = 32
_NBANDS = _BATCH // _BAND
_CHUNK = 25088


def _mm_kernel(x_ref, ft_ref, o_ref):
    x = x_ref[...]
    nrm = jnp.sqrt(jnp.sum(x * x, axis=1, keepdims=True))
    xn = x * (_INV_TEMP / jnp.clip(nrm, 1e-12, None))
    c = 0
    while c < _NUM_SAMPLES:
        w = min(_CHUNK, _NUM_SAMPLES - c)
        o_ref[:, c:c + w] = jax.lax.dot_general(
            xn, ft_ref[:, c:c + w], (((1,), (0,)), ((), ())),
            preferred_element_type=jnp.float32)
        c += _CHUNK


def kernel(inputs, targets, features):
    del targets
    features_t = jnp.swapaxes(features, 0, 1)
    return pl.pallas_call(
        _mm_kernel,
        grid=(_NBANDS,),
        in_specs=[
            pl.BlockSpec((_BAND, _NUM_FEATURES), lambda i: (i, 0)),
            pl.BlockSpec(memory_space=pltpu.MemorySpace.VMEM),
        ],
        out_specs=pl.BlockSpec((_BAND, _NUM_SAMPLES), lambda i: (i, 0)),
        out_shape=jax.ShapeDtypeStruct((_BATCH, _NUM_SAMPLES), jnp.float32),
        compiler_params=pltpu.CompilerParams(
            dimension_semantics=("arbitrary",)),
    )(inputs, features_t)
